# baseline (speedup 1.0000x reference)
import functools

import jax
import jax.numpy as jnp
from jax import lax
from jax.experimental import pallas as pl
from jax.experimental.pallas import tpu as pltpu
from jax.experimental.pallas import tpu_sc as plsc

N = 10000
E = 320000
IN_DIM = 128
HID = 256
NCLS = 10

NC = 2
NS = 16
NW = NC * NS
K = 128
NP = 10240
CHUNKS = 80
CH = 160
EP = NW * K * CHUNKS
EPW = CHUNKS * K
D = 128
RPT = NP // NS
DEGW = 16

_mesh = plsc.VectorSubcoreMesh(core_axis_name="c", subcore_axis_name="s")


@functools.partial(
    pl.kernel,
    out_type=jax.ShapeDtypeStruct((NP, D), jnp.float32),
    mesh=_mesh,
    scratch_types=(
        pltpu.VMEM((2, K), jnp.int32),
        pltpu.VMEM((2, K), jnp.int32),
        pltpu.VMEM((2, K), jnp.int32),
        pltpu.VMEM((2, K), jnp.int32),
        pltpu.VMEM((K, D), jnp.float32),
        pltpu.VMEM((K, D), jnp.float32),
        pltpu.VMEM_SHARED((NP, D), jnp.float32),
        pltpu.SemaphoreType.DMA,
        pltpu.SemaphoreType.DMA,
        pltpu.SemaphoreType.DMA,
        pltpu.SemaphoreType.DMA,
        pltpu.SemaphoreType.DMA,
        pltpu.SemaphoreType.DMA,
    ),
)
def _prop_kernel(table, eidx, zeros_h, out,
                 eb0, eb1, eb2, eb3, buf0, buf1, acc,
                 se0, se1, se2, se3, sg0, sg1):
    c = lax.axis_index("c")
    s = lax.axis_index("s")
    ebufs = (eb0, eb1, eb2, eb3)
    sems_e = (se0, se1, se2, se3)
    bufs = (buf0, buf1)
    sems_g = (sg0, sg1)
    base = s * CH

    @pl.when(c == 0)
    def _core0_body():
        pltpu.sync_copy(zeros_h, buf0)
        for j in range(RPT // K):
            pltpu.sync_copy(buf0, acc.at[pl.ds(s * RPT + j * K, K)])
        plsc.subcore_barrier()

        def _drain_g(b):
            pltpu.make_async_copy(table.at[pl.ds(0, K)], bufs[b],
                                  sems_g[b]).wait()

        def _drain_e(e):
            pltpu.make_async_copy(eidx.at[0], ebufs[e], sems_e[e]).wait()

        for e in range(4):
            pltpu.async_copy(eidx.at[base + e], ebufs[e], sems_e[e])
        for b in range(2):
            _drain_e(b)
            pltpu.async_copy(table.at[ebufs[b].at[0]], bufs[b], sems_g[b])

        @pl.loop(0, CH, step=4)
        def _body(j):
            for i in range(4):
                ch = j + i
                b = i % 2
                _drain_g(b)
                pltpu.sync_copy(bufs[b], acc.at[ebufs[i].at[1]], add=True)
                pltpu.async_copy(eidx.at[base + ch + 4], ebufs[i], sems_e[i])
                _drain_e((i + 2) % 4)
                pltpu.async_copy(table.at[ebufs[(i + 2) % 4].at[0]],
                                 bufs[b], sems_g[b])

        _drain_g(0)
        _drain_g(1)
        _drain_e(2)
        _drain_e(3)

        plsc.subcore_barrier()
        pltpu.sync_copy(acc.at[pl.ds(s * RPT, RPT)],
                        out.at[pl.ds(s * RPT, RPT)])


R = 1024
G = NP // R


def _scale_body(degs_ref, degd_ref, x_ref, xs_ref, ns_ref, nd_ref):
    d_out = degs_ref[:, :DEGW]
    d_in = degd_ref[:, :DEGW]
    ns = lax.rsqrt(jnp.maximum(d_out, 1.0))
    nd = lax.rsqrt(jnp.maximum(d_in, 1.0))
    ns_ref[...] = ns
    nd_ref[...] = nd
    xs_ref[...] = x_ref[...] * ns[:, 0:1]


def _l1_body(agg_ref, nd_ref, ns_ref, w_ref, b_ref, oa_ref, ob_ref):
    agg = agg_ref[...] * nd_ref[:, 0:1]
    h = jnp.dot(agg, w_ref[...], preferred_element_type=jnp.float32) + b_ref[...]
    h = jnp.maximum(h, 0.0) * ns_ref[:, 0:1]
    oa_ref[...] = h[:, :D]
    ob_ref[...] = h[:, D:]


def _l2_body(agga_ref, aggb_ref, nd_ref, w_ref, b_ref, o_ref):
    a = jnp.concatenate([agga_ref[...], aggb_ref[...]], axis=1)
    a = a * nd_ref[:, 0:1]
    h = jnp.dot(a, w_ref[...], preferred_element_type=jnp.float32) + b_ref[...]
    o_ref[...] = jnp.maximum(h, 0.0)


def _pool_body(h2_ref, gwt_ref, gb_ref, w1_ref, b1_ref, w2_ref, b2_ref,
               out_ref, gate_ref, hg_ref):
    h2 = h2_ref[...]
    logits = jnp.sum(h2 * gwt_ref[...], axis=1, keepdims=True) + gb_ref[0, 0]
    rows = lax.broadcasted_iota(jnp.int32, (NP, 1), 0)
    valid = rows < N
    ml = jnp.where(valid, logits, -1e30)
    m = jnp.max(ml)
    e = jnp.where(valid, jnp.exp(ml - m), 0.0)
    gate = e / jnp.sum(e)
    gate_ref[...] = gate
    hg = jnp.sum(gate * h2, axis=0, keepdims=True)
    hg_ref[...] = hg
    a2 = jnp.dot(hg, w1_ref[...], preferred_element_type=jnp.float32) + b1_ref[...]
    a3 = jnp.dot(a2, w2_ref[...], preferred_element_type=jnp.float32) + b2_ref[...]
    out_ref[...] = jax.nn.sigmoid(a3)


def kernel(x, edge_index, W1, b1, W2, b2, gate_w, gate_b, cls1_w, cls1_b,
           cls2_w, cls2_b):
    f32 = jnp.float32
    src = edge_index[0]
    dst = edge_index[1]
    pad_idx = jnp.full((EP - E,), N, jnp.int32)
    srcp = jnp.concatenate([src, pad_idx]).reshape(NW * CHUNKS, K)
    dstp = jnp.concatenate([dst, pad_idx]).reshape(NW * CHUNKS, K)
    pad_rows = jnp.full((4, 2, K), N, jnp.int32)
    eidx = jnp.concatenate([jnp.stack([srcp, dstp], axis=1), pad_rows])
    zrow = jnp.zeros_like(srcp)
    eidx_degs = jnp.concatenate([jnp.stack([zrow, srcp], axis=1), pad_rows])
    eidx_degd = jnp.concatenate([jnp.stack([zrow, dstp], axis=1), pad_rows])
    xp = jnp.pad(x, ((0, NP - N), (0, 0)))

    zeros_kd = jnp.zeros((K, D), f32)

    ones_tab = jnp.concatenate([jnp.ones((N, D), f32),
                                jnp.zeros((NP - N, D), f32)])
    degs3 = _prop_kernel(ones_tab, eidx_degs, zeros_kd)
    degd3 = _prop_kernel(ones_tab, eidx_degd, zeros_kd)

    xs, ns, nd = pl.pallas_call(
        _scale_body,
        grid=(G,),
        in_specs=[pl.BlockSpec((R, D), lambda i: (i, 0)),
                  pl.BlockSpec((R, D), lambda i: (i, 0)),
                  pl.BlockSpec((R, IN_DIM), lambda i: (i, 0))],
        out_specs=[pl.BlockSpec((R, IN_DIM), lambda i: (i, 0)),
                   pl.BlockSpec((R, DEGW), lambda i: (i, 0)),
                   pl.BlockSpec((R, DEGW), lambda i: (i, 0))],
        out_shape=[jax.ShapeDtypeStruct((NP, IN_DIM), f32),
                   jax.ShapeDtypeStruct((NP, DEGW), f32),
                   jax.ShapeDtypeStruct((NP, DEGW), f32)],
    )(degs3, degd3, xp)

    p1 = _prop_kernel(xs, eidx, zeros_kd)

    h1a, h1b = pl.pallas_call(
        _l1_body,
        grid=(G,),
        in_specs=[pl.BlockSpec((R, D), lambda i: (i, 0)),
                  pl.BlockSpec((R, DEGW), lambda i: (i, 0)),
                  pl.BlockSpec((R, DEGW), lambda i: (i, 0)),
                  pl.BlockSpec((IN_DIM, HID), lambda i: (0, 0)),
                  pl.BlockSpec((1, HID), lambda i: (0, 0))],
        out_specs=[pl.BlockSpec((R, D), lambda i: (i, 0)),
                   pl.BlockSpec((R, D), lambda i: (i, 0))],
        out_shape=[jax.ShapeDtypeStruct((NP, D), f32),
                   jax.ShapeDtypeStruct((NP, D), f32)],
    )(p1, nd, ns, W1, b1.reshape(1, HID))

    p2a = _prop_kernel(h1a, eidx, zeros_kd)
    p2b = _prop_kernel(h1b, eidx, zeros_kd)

    h2 = pl.pallas_call(
        _l2_body,
        grid=(G,),
        in_specs=[pl.BlockSpec((R, D), lambda i: (i, 0)),
                  pl.BlockSpec((R, D), lambda i: (i, 0)),
                  pl.BlockSpec((R, DEGW), lambda i: (i, 0)),
                  pl.BlockSpec((HID, HID), lambda i: (0, 0)),
                  pl.BlockSpec((1, HID), lambda i: (0, 0))],
        out_specs=pl.BlockSpec((R, HID), lambda i: (i, 0)),
        out_shape=jax.ShapeDtypeStruct((NP, HID), f32),
    )(p2a, p2b, nd, W2, b2.reshape(1, HID))

    gwt = gate_w.reshape(1, HID)
    gb = gate_b.reshape(1, 1)
    cls2_wp = jnp.pad(cls2_w, ((0, 0), (0, 128 - NCLS)))
    cls2_bp = jnp.pad(cls2_b, (0, 128 - NCLS)).reshape(1, 128)

    outp, gate_full, hg = pl.pallas_call(
        _pool_body,
        out_shape=[jax.ShapeDtypeStruct((1, 128), f32),
                   jax.ShapeDtypeStruct((NP, 1), f32),
                   jax.ShapeDtypeStruct((1, HID), f32)],
    )(h2, gwt, gb, cls1_w, cls1_b.reshape(1, HID), cls2_wp, cls2_bp)

    return (outp[:, :NCLS], gate_full[:N], hg)


# speedup vs baseline: 11.3920x; 11.3920x over previous
import functools

import jax
import jax.numpy as jnp
from jax import lax
from jax.experimental import pallas as pl
from jax.experimental.pallas import tpu as pltpu
from jax.experimental.pallas import tpu_sc as plsc

N = 10000
E = 320000
IN_DIM = 128
HID = 256
NCLS = 10

NC = 2
NS = 16
NW = NC * NS
K = 128
NP = 10240
CHUNKS = 80
CH_FAST = 128
CH_SLOW = 32
EP = NW * K * CHUNKS
EPW = CHUNKS * K
D = 128
RPT = NP // NS
DEGW = 16

_mesh = plsc.VectorSubcoreMesh(core_axis_name="c", subcore_axis_name="s")


@functools.partial(
    pl.kernel,
    out_type=jax.ShapeDtypeStruct((NC * NP, D), jnp.float32),
    mesh=_mesh,
    scratch_types=(
        pltpu.VMEM((2, K), jnp.int32),
        pltpu.VMEM((2, K), jnp.int32),
        pltpu.VMEM((2, K), jnp.int32),
        pltpu.VMEM((2, K), jnp.int32),
        pltpu.VMEM((K, D), jnp.float32),
        pltpu.VMEM((K, D), jnp.float32),
        pltpu.VMEM_SHARED((NP, D), jnp.float32),
        pltpu.SemaphoreType.DMA,
        pltpu.SemaphoreType.DMA,
        pltpu.SemaphoreType.DMA,
        pltpu.SemaphoreType.DMA,
        pltpu.SemaphoreType.DMA,
        pltpu.SemaphoreType.DMA,
    ),
)
def _prop_kernel(table, eidx, zeros_h, out,
                 eb0, eb1, eb2, eb3, buf0, buf1, acc,
                 se0, se1, se2, se3, sg0, sg1):
    c = lax.axis_index("c")
    s = lax.axis_index("s")
    ebufs = (eb0, eb1, eb2, eb3)
    sems_e = (se0, se1, se2, se3)
    bufs = (buf0, buf1)
    sems_g = (sg0, sg1)
    nch = jnp.where(c == 0, CH_FAST, CH_SLOW)
    base = jnp.where(c == 0, s * CH_FAST, NS * CH_FAST + s * CH_SLOW)

    pltpu.sync_copy(zeros_h, buf0)
    for j in range(RPT // K):
        pltpu.sync_copy(buf0, acc.at[pl.ds(s * RPT + j * K, K)])
    plsc.subcore_barrier()

    def _drain_g(b):
        pltpu.make_async_copy(table.at[pl.ds(0, K)], bufs[b], sems_g[b]).wait()

    def _drain_e(e):
        pltpu.make_async_copy(eidx.at[0], ebufs[e], sems_e[e]).wait()

    for e in range(4):
        pltpu.async_copy(eidx.at[base + e], ebufs[e], sems_e[e])
    for b in range(2):
        _drain_e(b)
        pltpu.async_copy(table.at[ebufs[b].at[0]], bufs[b], sems_g[b])

    @pl.loop(0, nch, step=4)
    def _body(j):
        for i in range(4):
            ch = j + i
            b = i % 2
            _drain_g(b)
            pltpu.sync_copy(bufs[b], acc.at[ebufs[i].at[1]], add=True)
            pltpu.async_copy(eidx.at[base + ch + 4], ebufs[i], sems_e[i])
            _drain_e((i + 2) % 4)
            pltpu.async_copy(table.at[ebufs[(i + 2) % 4].at[0]],
                             bufs[b], sems_g[b])

    _drain_g(0)
    _drain_g(1)
    _drain_e(2)
    _drain_e(3)

    plsc.subcore_barrier()
    pltpu.sync_copy(acc.at[pl.ds(s * RPT, RPT)],
                    out.at[pl.ds(c * NP + s * RPT, RPT)])


R = 1024
G = NP // R


def _scale_body(degs_ref, degd_ref, x_ref, xs_ref, ns_ref, nd_ref):
    d_out = degs_ref[0, :, :DEGW] + degs_ref[1, :, :DEGW]
    d_in = degd_ref[0, :, :DEGW] + degd_ref[1, :, :DEGW]
    ns = lax.rsqrt(jnp.maximum(d_out, 1.0))
    nd = lax.rsqrt(jnp.maximum(d_in, 1.0))
    ns_ref[...] = ns
    nd_ref[...] = nd
    xs_ref[...] = x_ref[...] * ns[:, 0:1]


def _l1_body(agg_ref, nd_ref, ns_ref, w_ref, b_ref, oa_ref, ob_ref):
    agg = (agg_ref[0] + agg_ref[1]) * nd_ref[:, 0:1]
    h = jnp.dot(agg, w_ref[...], preferred_element_type=jnp.float32) + b_ref[...]
    h = jnp.maximum(h, 0.0) * ns_ref[:, 0:1]
    oa_ref[...] = h[:, :D]
    ob_ref[...] = h[:, D:]


def _l2_body(agga_ref, aggb_ref, nd_ref, w_ref, b_ref, o_ref):
    a = jnp.concatenate(
        [agga_ref[0] + agga_ref[1], aggb_ref[0] + aggb_ref[1]], axis=1)
    a = a * nd_ref[:, 0:1]
    h = jnp.dot(a, w_ref[...], preferred_element_type=jnp.float32) + b_ref[...]
    o_ref[...] = jnp.maximum(h, 0.0)


def _pool_body(h2_ref, gwt_ref, gb_ref, w1_ref, b1_ref, w2_ref, b2_ref,
               out_ref, gate_ref, hg_ref):
    h2 = h2_ref[...]
    logits = jnp.sum(h2 * gwt_ref[...], axis=1, keepdims=True) + gb_ref[0, 0]
    rows = lax.broadcasted_iota(jnp.int32, (NP, 1), 0)
    valid = rows < N
    ml = jnp.where(valid, logits, -1e30)
    m = jnp.max(ml)
    e = jnp.where(valid, jnp.exp(ml - m), 0.0)
    gate = e / jnp.sum(e)
    gate_ref[...] = gate
    hg = jnp.sum(gate * h2, axis=0, keepdims=True)
    hg_ref[...] = hg
    a2 = jnp.dot(hg, w1_ref[...], preferred_element_type=jnp.float32) + b1_ref[...]
    a3 = jnp.dot(a2, w2_ref[...], preferred_element_type=jnp.float32) + b2_ref[...]
    out_ref[...] = jax.nn.sigmoid(a3)


def kernel(x, edge_index, W1, b1, W2, b2, gate_w, gate_b, cls1_w, cls1_b,
           cls2_w, cls2_b):
    f32 = jnp.float32
    src = edge_index[0]
    dst = edge_index[1]
    pad_idx = jnp.full((EP - E,), N, jnp.int32)
    srcp = jnp.concatenate([src, pad_idx]).reshape(NW * CHUNKS, K)
    dstp = jnp.concatenate([dst, pad_idx]).reshape(NW * CHUNKS, K)
    pad_rows = jnp.full((4, 2, K), N, jnp.int32)
    eidx = jnp.concatenate([jnp.stack([srcp, dstp], axis=1), pad_rows])
    eidx_rev = jnp.concatenate([jnp.stack([dstp, srcp], axis=1), pad_rows])
    xp = jnp.pad(x, ((0, NP - N), (0, 0)))

    zeros_kd = jnp.zeros((K, D), f32)

    ones_tab = jnp.concatenate([jnp.ones((N, D), f32),
                                jnp.zeros((NP - N, D), f32)])
    degs3 = _prop_kernel(ones_tab, eidx_rev, zeros_kd).reshape(2, NP, D)
    degd3 = _prop_kernel(ones_tab, eidx, zeros_kd).reshape(2, NP, D)

    xs, ns, nd = pl.pallas_call(
        _scale_body,
        grid=(G,),
        in_specs=[pl.BlockSpec((2, R, D), lambda i: (0, i, 0)),
                  pl.BlockSpec((2, R, D), lambda i: (0, i, 0)),
                  pl.BlockSpec((R, IN_DIM), lambda i: (i, 0))],
        out_specs=[pl.BlockSpec((R, IN_DIM), lambda i: (i, 0)),
                   pl.BlockSpec((R, DEGW), lambda i: (i, 0)),
                   pl.BlockSpec((R, DEGW), lambda i: (i, 0))],
        out_shape=[jax.ShapeDtypeStruct((NP, IN_DIM), f32),
                   jax.ShapeDtypeStruct((NP, DEGW), f32),
                   jax.ShapeDtypeStruct((NP, DEGW), f32)],
    )(degs3, degd3, xp)

    p1 = _prop_kernel(xs, eidx, zeros_kd).reshape(2, NP, D)

    h1a, h1b = pl.pallas_call(
        _l1_body,
        grid=(G,),
        in_specs=[pl.BlockSpec((2, R, D), lambda i: (0, i, 0)),
                  pl.BlockSpec((R, DEGW), lambda i: (i, 0)),
                  pl.BlockSpec((R, DEGW), lambda i: (i, 0)),
                  pl.BlockSpec((IN_DIM, HID), lambda i: (0, 0)),
                  pl.BlockSpec((1, HID), lambda i: (0, 0))],
        out_specs=[pl.BlockSpec((R, D), lambda i: (i, 0)),
                   pl.BlockSpec((R, D), lambda i: (i, 0))],
        out_shape=[jax.ShapeDtypeStruct((NP, D), f32),
                   jax.ShapeDtypeStruct((NP, D), f32)],
    )(p1, nd, ns, W1, b1.reshape(1, HID))

    p2a = _prop_kernel(h1a, eidx, zeros_kd).reshape(2, NP, D)
    p2b = _prop_kernel(h1b, eidx, zeros_kd).reshape(2, NP, D)

    h2 = pl.pallas_call(
        _l2_body,
        grid=(G,),
        in_specs=[pl.BlockSpec((2, R, D), lambda i: (0, i, 0)),
                  pl.BlockSpec((2, R, D), lambda i: (0, i, 0)),
                  pl.BlockSpec((R, DEGW), lambda i: (i, 0)),
                  pl.BlockSpec((HID, HID), lambda i: (0, 0)),
                  pl.BlockSpec((1, HID), lambda i: (0, 0))],
        out_specs=pl.BlockSpec((R, HID), lambda i: (i, 0)),
        out_shape=jax.ShapeDtypeStruct((NP, HID), f32),
    )(p2a, p2b, nd, W2, b2.reshape(1, HID))

    gwt = gate_w.reshape(1, HID)
    gb = gate_b.reshape(1, 1)
    cls2_wp = jnp.pad(cls2_w, ((0, 0), (0, 128 - NCLS)))
    cls2_bp = jnp.pad(cls2_b, (0, 128 - NCLS)).reshape(1, 128)

    outp, gate_full, hg = pl.pallas_call(
        _pool_body,
        out_shape=[jax.ShapeDtypeStruct((1, 128), f32),
                   jax.ShapeDtypeStruct((NP, 1), f32),
                   jax.ShapeDtypeStruct((1, HID), f32)],
    )(h2, gwt, gb, cls1_w, cls1_b.reshape(1, HID), cls2_wp, cls2_bp)

    return (outp[:, :NCLS], gate_full[:N], hg)
